# Initial kernel scaffold; baseline (speedup 1.0000x reference)
import jax
import jax.numpy as jnp
from jax.experimental import pallas as pl


def kernel(pos, node_attrs, atom_emb, gamma, fc1_w, fc1_b, fc2_w, fc2_b, w_self, w_readout):
    raise NotImplementedError("write your pallas kernel here")


# speedup vs baseline: 33.4478x; 33.4478x over previous
import math

import jax
import jax.numpy as jnp
from jax.experimental import pallas as pl

B, N = 32, 32
NUM_BASIS = 20
R_MAX = 10.0
D_EMB = 32
D_SCAL = 64
_C = 1.0 / math.sqrt(D_EMB)
BPS = 4
P = BPS * N * N


def _tc_body(pos_ref, na_ref, ae_exp_ref, fc2w1_ref, fc2b1_ref, fc1wT_ref,
             fc1b_ref, gamma_ref, wself_ref, wread_ref, atom_ref, out_ref):
    v = jnp.dot(ae_exp_ref[...], fc2w1_ref[...])
    s0 = jnp.dot(ae_exp_ref[...], fc2b1_ref[...])
    q = jnp.dot(wself_ref[...], wread_ref[...])
    aeq = jnp.dot(atom_ref[...], q) * _C
    w3 = (_C * (N - 1)) * s0 + aeq

    pos = pos_ref[...]
    na = na_ref[...]

    pi = jnp.broadcast_to(pos[:, :, None, :], (BPS, N, N, 3)).reshape(P, 3)
    pj = jnp.broadcast_to(pos[:, None, :, :], (BPS, N, N, 3)).reshape(P, 3)
    diff = pi - pj
    d2 = jnp.sum(diff * diff, axis=1, keepdims=True)
    d = jnp.sqrt(jnp.maximum(d2, 0.0))

    centers = jax.lax.broadcasted_iota(jnp.int32, (1, NUM_BASIS), 1).astype(
        jnp.float32) * (R_MAX / (NUM_BASIS - 1))
    g = gamma_ref[...]
    t = d - centers
    rbf = jnp.exp(-g * t * t)
    hid = jnp.maximum(jnp.dot(rbf, fc1wT_ref[...]) + fc1b_ref[...], 0.0)

    a0 = na[:, :, 0:1]
    a1 = na[:, :, 1:2]
    a2 = na[:, :, 2:3]
    oh0 = ((a0 >= a1) & (a0 >= a2)).astype(jnp.float32)
    oh1 = ((a1 > a0) & (a1 >= a2)).astype(jnp.float32)
    oh2 = ((a2 > a0) & (a2 > a1)).astype(jnp.float32)
    oh = jnp.concatenate([oh0, oh1, oh2], axis=2)

    ohj = jnp.broadcast_to(oh[:, None, :, :], (BPS, N, N, 3)).reshape(P, 3)
    vz = jnp.dot(ohj, v)
    rd = jnp.sum(hid * vz, axis=1, keepdims=True)

    r = jax.lax.broadcasted_iota(jnp.int32, (P, 1), 0)
    ii = (r // N) % N
    jj = r % N
    rd = jnp.where(ii != jj, rd, 0.0)

    bsel = (r // (N * N) == jax.lax.broadcasted_iota(
        jnp.int32, (1, BPS), 1)).astype(jnp.float32)
    pair_b = jax.lax.dot_general(bsel, rd, (((0,), (0,)), ((), ())))

    node_term = jnp.dot(oh.reshape(BPS * N, 3), w3)
    rn = jax.lax.broadcasted_iota(jnp.int32, (BPS * N, 1), 0)
    nsel = (rn // N == jax.lax.broadcasted_iota(
        jnp.int32, (1, BPS), 1)).astype(jnp.float32)
    node_b = jax.lax.dot_general(nsel, node_term, (((0,), (0,)), ((), ())))

    out_ref[...] = (0.125 * (_C * pair_b + node_b))[None]


def kernel(pos, node_attrs, atom_emb, gamma, fc1_w, fc1_b, fc2_w, fc2_b,
           w_self, w_readout):
    ae_exp = (atom_emb[:, :, None] * w_readout[None, None, :, 0]).reshape(
        3, D_EMB * D_SCAL)
    fc2w1 = fc2_w[:D_EMB * D_SCAL, :]
    fc2b1 = fc2_b[:D_EMB * D_SCAL].reshape(D_EMB * D_SCAL, 1)
    fc1wT = fc1_w.T
    fc1b = fc1_b.reshape(1, 32)
    gamma2 = jnp.asarray(gamma, jnp.float32).reshape(1, 1)

    grid = (B // BPS,)
    full = lambda shape: pl.BlockSpec(shape, lambda b: (0,) * len(shape))
    out = pl.pallas_call(
        _tc_body,
        grid=grid,
        in_specs=[
            pl.BlockSpec((BPS, N, 3), lambda b: (b, 0, 0)),
            pl.BlockSpec((BPS, N, 3), lambda b: (b, 0, 0)),
            full((3, D_EMB * D_SCAL)),
            full((D_EMB * D_SCAL, 32)),
            full((D_EMB * D_SCAL, 1)),
            full((NUM_BASIS, 32)),
            full((1, 32)),
            full((1, 1)),
            full((D_EMB, D_SCAL)),
            full((D_SCAL, 1)),
            full((3, D_EMB)),
        ],
        out_specs=pl.BlockSpec((1, BPS, 1), lambda b: (b, 0, 0)),
        out_shape=jax.ShapeDtypeStruct((B // BPS, BPS, 1), jnp.float32),
    )(pos, node_attrs, ae_exp, fc2w1, fc2b1, fc1wT, fc1b, gamma2, w_self,
      w_readout, atom_emb)
    return out.reshape(B, 1)
